# baseline (speedup 1.0000x reference)
import jax
import jax.numpy as jnp
from jax.experimental import pallas as pl

_ROWS = 4096
_COLS = 8192
_BLOCK_ROWS = 128


def _copy_body(in_ref, out_ref):
    out_ref[...] = in_ref[...]


def kernel(inputs, w):
    del inputs
    x = w.reshape(_ROWS, _COLS)
    y = pl.pallas_call(
        _copy_body,
        grid=(_ROWS // _BLOCK_ROWS,),
        in_specs=[pl.BlockSpec((_BLOCK_ROWS, _COLS), lambda i: (i, 0))],
        out_specs=pl.BlockSpec((_BLOCK_ROWS, _COLS), lambda i: (i, 0)),
        out_shape=jax.ShapeDtypeStruct((_ROWS, _COLS), jnp.float32),
    )(x)
    return y.reshape(_ROWS, 4096, 2)


# speedup vs baseline: 9.9742x; 9.9742x over previous
import jax
import jax.numpy as jnp
from jax.experimental import pallas as pl

_N = 4096
_BLOCK_ROWS = 128


def _copy_body(in_ref, out_ref):
    out_ref[...] = in_ref[...]


def kernel(inputs, w):
    del inputs
    x = jnp.transpose(w, (0, 2, 1))
    y = pl.pallas_call(
        _copy_body,
        grid=(_N // _BLOCK_ROWS,),
        in_specs=[pl.BlockSpec((_BLOCK_ROWS, 2, _N), lambda i: (i, 0, 0))],
        out_specs=pl.BlockSpec((_BLOCK_ROWS, 2, _N), lambda i: (i, 0, 0)),
        out_shape=jax.ShapeDtypeStruct((_N, 2, _N), jnp.float32),
    )(x)
    return jnp.transpose(y, (0, 2, 1))
